# baseline (speedup 1.0000x reference)
import jax
import jax.numpy as jnp
from jax import lax
from jax.experimental import pallas as pl
from jax.experimental.pallas import tpu as pltpu
from jax.experimental.pallas import tpu_sc as plsc

D_MODEL = 1024
D_FF = 1024
E = 64
T = 2048
BT = 64
MAXTILES = T // BT + E
SLOTS = MAXTILES * BT
_CH = 128
_NC = T // _CH
_NWORK = 32
_ROWS_W = T // _NWORK


def _fiota(shape, dim):
    return jax.lax.broadcasted_iota(jnp.int32, shape, dim).astype(jnp.float32)


def _prep_kernel(x_ref, wr_ref, te_ref, q_ref):
    x = x_ref[...]
    wr = wr_ref[...]
    logits = jax.lax.dot_general(
        x, wr, (((1,), (1,)), ((), ())),
        preferred_element_type=jnp.float32)
    m = jnp.max(logits, axis=1, keepdims=True)
    iota_e = _fiota((T, E), 1)
    e_tok = jnp.min(jnp.where(logits == m, iota_e, float(E)), axis=1,
                    keepdims=True)
    oh = jnp.where(iota_e == e_tok, 1.0, 0.0)
    counts = jnp.sum(oh, axis=0, keepdims=True)
    nt = jnp.floor((counts + (BT - 1)) * (1.0 / BT))
    ue = jnp.where(_fiota((E, E), 0) <= _fiota((E, E), 1), 1.0, 0.0)
    cumt = jax.lax.dot_general(
        nt, ue, (((1,), (0,)), ((), ())),
        preferred_element_type=jnp.float32)
    po = (cumt - nt) * BT

    lt = jnp.where(_fiota((_CH, _CH), 0) > _fiota((_CH, _CH), 1), 1.0, 0.0)

    base = jnp.zeros((1, E), jnp.float32)
    for c in range(_NC):
        ohc = oh[c * _CH:(c + 1) * _CH, :]
        within = jax.lax.dot_general(
            lt, ohc, (((1,), (0,)), ((), ())),
            preferred_element_type=jnp.float32)
        rank_c = jnp.sum((within + base) * ohc, axis=1)
        po_c = jnp.sum(ohc * po, axis=1)
        q_ref[0, c * _CH:(c + 1) * _CH] = (po_c + rank_c).astype(jnp.int32)
        base = base + jnp.sum(ohc, axis=0, keepdims=True)

    jt = _fiota((MAXTILES, E), 0)
    te = jnp.sum(jnp.where(cumt <= jt, 1.0, 0.0), axis=1)
    te_ref[0, :] = jnp.minimum(te, float(E - 1)).astype(jnp.int32)


def _gemm_kernel(te_ref, q_ref, x_ref, w1_ref, v1_ref, w2_ref, out_ref):
    j = pl.program_id(0)
    qv = q_ref[0, :]
    slot = j * BT + jax.lax.broadcasted_iota(jnp.int32, (BT, T), 0)
    g = jnp.where(qv[None, :] == slot, 1.0, 0.0)
    xt = jax.lax.dot_general(
        g, x_ref[...], (((1,), (0,)), ((), ())),
        preferred_element_type=jnp.float32)
    h1 = jax.lax.dot_general(
        xt, w1_ref[0], (((1,), (1,)), ((), ())),
        preferred_element_type=jnp.float32)
    hv = jax.lax.dot_general(
        xt, v1_ref[0], (((1,), (1,)), ((), ())),
        preferred_element_type=jnp.float32)
    h = h1 * jax.lax.logistic(h1) * hv
    out_ref[...] = jax.lax.dot_general(
        h, w2_ref[0], (((1,), (1,)), ((), ())),
        preferred_element_type=jnp.float32)


def _sc_unpermute_body(outs_ref, q_ref, out_ref, idx_v, rows_v, sem):
    wid = lax.axis_index("s") * 2 + lax.axis_index("c")
    b = wid * _ROWS_W
    pltpu.sync_copy(q_ref.at[pl.ds(b, _ROWS_W)], idx_v)
    pltpu.async_copy(outs_ref.at[idx_v], rows_v, sem).wait()
    pltpu.sync_copy(rows_v, out_ref.at[pl.ds(b, _ROWS_W)])


def _sc_mesh():
    return plsc.VectorSubcoreMesh(core_axis_name="c", subcore_axis_name="s")
_SC_SCRATCH = [
    pltpu.VMEM((_ROWS_W,), jnp.int32),
    pltpu.VMEM((_ROWS_W, D_MODEL), jnp.float32),
    pltpu.SemaphoreType.DMA,
]


def kernel(hidden_states, w_router, w1, v1, w2):
    orig_shape = hidden_states.shape
    x = hidden_states.reshape(T, D_MODEL)

    te, q = pl.pallas_call(
        _prep_kernel,
        out_shape=(
            jax.ShapeDtypeStruct((1, MAXTILES), jnp.int32),
            jax.ShapeDtypeStruct((1, T), jnp.int32),
        ),
    )(x, w_router)
    qf = q.reshape(T)

    grid_spec = pltpu.PrefetchScalarGridSpec(
        num_scalar_prefetch=1,
        grid=(MAXTILES,),
        in_specs=[
            pl.BlockSpec((1, T), lambda j, te_s: (0, 0)),
            pl.BlockSpec((T, D_MODEL), lambda j, te_s: (0, 0)),
            pl.BlockSpec((1, D_FF, D_MODEL), lambda j, te_s: (te_s[0, j], 0, 0)),
            pl.BlockSpec((1, D_FF, D_MODEL), lambda j, te_s: (te_s[0, j], 0, 0)),
            pl.BlockSpec((1, D_MODEL, D_FF), lambda j, te_s: (te_s[0, j], 0, 0)),
        ],
        out_specs=pl.BlockSpec((BT, D_MODEL), lambda j, te_s: (j, 0)),
    )
    out_s = pl.pallas_call(
        _gemm_kernel,
        grid_spec=grid_spec,
        out_shape=jax.ShapeDtypeStruct((SLOTS, D_MODEL), jnp.float32),
    )(te, q, x, w1, v1, w2)

    out = pl.kernel(
        _sc_unpermute_body,
        out_type=jax.ShapeDtypeStruct((T, D_MODEL), jnp.float32),
        mesh=_sc_mesh(),
        scratch_types=_SC_SCRATCH,
    )(out_s, qf)
    return out.reshape(orig_shape)


# speedup vs baseline: 1.1313x; 1.1313x over previous
import jax
import jax.numpy as jnp
from jax import lax
from jax.experimental import pallas as pl
from jax.experimental.pallas import tpu as pltpu
from jax.experimental.pallas import tpu_sc as plsc

D_MODEL = 1024
D_FF = 1024
E = 64
T = 2048
BT = 128
MAXTILES = T // BT + E
SLOTS = MAXTILES * BT
_CH = 128
_NC = T // _CH
_NWORK = 32
_ROWS_W = T // _NWORK


def _fiota(shape, dim):
    return jax.lax.broadcasted_iota(jnp.int32, shape, dim).astype(jnp.float32)


def _prep_kernel(x_ref, wr_ref, te_ref, q_ref):
    x = x_ref[...]
    wr = wr_ref[...]
    logits = jax.lax.dot_general(
        x, wr, (((1,), (1,)), ((), ())),
        preferred_element_type=jnp.float32)
    m = jnp.max(logits, axis=1, keepdims=True)
    iota_e = _fiota((T, E), 1)
    e_tok = jnp.min(jnp.where(logits == m, iota_e, float(E)), axis=1,
                    keepdims=True)
    oh = jnp.where(iota_e == e_tok, 1.0, 0.0)
    counts = jnp.sum(oh, axis=0, keepdims=True)
    nt = jnp.floor((counts + (BT - 1)) * (1.0 / BT))
    ue = jnp.where(_fiota((E, E), 0) <= _fiota((E, E), 1), 1.0, 0.0)
    cumt = jax.lax.dot_general(
        nt, ue, (((1,), (0,)), ((), ())),
        preferred_element_type=jnp.float32)
    po = (cumt - nt) * BT

    lt = jnp.where(_fiota((_CH, _CH), 0) > _fiota((_CH, _CH), 1), 1.0, 0.0)

    base = jnp.zeros((1, E), jnp.float32)
    for c in range(_NC):
        ohc = oh[c * _CH:(c + 1) * _CH, :]
        within = jax.lax.dot_general(
            lt, ohc, (((1,), (0,)), ((), ())),
            preferred_element_type=jnp.float32)
        rank_c = jnp.sum((within + base) * ohc, axis=1)
        po_c = jnp.sum(ohc * po, axis=1)
        q_ref[0, c * _CH:(c + 1) * _CH] = (po_c + rank_c).astype(jnp.int32)
        base = base + jnp.sum(ohc, axis=0, keepdims=True)

    jt = _fiota((MAXTILES, E), 0)
    te = jnp.sum(jnp.where(cumt <= jt, 1.0, 0.0), axis=1)
    te_ref[0, :] = jnp.minimum(te, float(E - 1)).astype(jnp.int32)


def _gemm_kernel(te_ref, xs_ref, w1_ref, v1_ref, w2_ref, out_ref):
    xt = xs_ref[...]
    h1 = jax.lax.dot_general(
        xt, w1_ref[0], (((1,), (1,)), ((), ())),
        preferred_element_type=jnp.float32)
    hv = jax.lax.dot_general(
        xt, v1_ref[0], (((1,), (1,)), ((), ())),
        preferred_element_type=jnp.float32)
    h = h1 * jax.lax.logistic(h1) * hv
    out_ref[...] = jax.lax.dot_general(
        h, w2_ref[0], (((1,), (1,)), ((), ())),
        preferred_element_type=jnp.float32)


_HW = _ROWS_W // 2


def _sc_dispatch_body(x_ref, q_ref, xs_ref, idx_v, rows_v, sem, sem2):
    wid = lax.axis_index("s") * 2 + lax.axis_index("c")
    b = wid * _ROWS_W
    cp_i = pltpu.async_copy(q_ref.at[pl.ds(b, _ROWS_W)], idx_v, sem)
    cp0 = pltpu.async_copy(x_ref.at[pl.ds(b, _HW)], rows_v.at[pl.ds(0, _HW)],
                           sem2)
    cp_i.wait()
    cp0.wait()
    sc0 = pltpu.async_copy(rows_v.at[pl.ds(0, _HW)],
                           xs_ref.at[idx_v.at[pl.ds(0, _HW)]], sem)
    cp1 = pltpu.async_copy(x_ref.at[pl.ds(b + _HW, _HW)],
                           rows_v.at[pl.ds(_HW, _HW)], sem2)
    cp1.wait()
    sc1 = pltpu.async_copy(rows_v.at[pl.ds(_HW, _HW)],
                           xs_ref.at[idx_v.at[pl.ds(_HW, _HW)]], sem)
    sc0.wait()
    sc1.wait()


def _sc_unpermute_body(outs_ref, q_ref, out_ref, idx_v, rows_v, sem, sem2):
    wid = lax.axis_index("s") * 2 + lax.axis_index("c")
    b = wid * _ROWS_W
    pltpu.sync_copy(q_ref.at[pl.ds(b, _ROWS_W)], idx_v)
    pltpu.async_copy(outs_ref.at[idx_v], rows_v, sem).wait()
    pltpu.sync_copy(rows_v, out_ref.at[pl.ds(b, _ROWS_W)])


def _sc_mesh():
    return plsc.VectorSubcoreMesh(core_axis_name="c", subcore_axis_name="s")
_SC_SCRATCH = [
    pltpu.VMEM((_ROWS_W,), jnp.int32),
    pltpu.VMEM((_ROWS_W, D_MODEL), jnp.float32),
    pltpu.SemaphoreType.DMA,
    pltpu.SemaphoreType.DMA,
]


def kernel(hidden_states, w_router, w1, v1, w2):
    orig_shape = hidden_states.shape
    x = hidden_states.reshape(T, D_MODEL)

    te, q = pl.pallas_call(
        _prep_kernel,
        out_shape=(
            jax.ShapeDtypeStruct((1, MAXTILES), jnp.int32),
            jax.ShapeDtypeStruct((1, T), jnp.int32),
        ),
    )(x, w_router)
    qf = q.reshape(T)

    x_s = pl.kernel(
        _sc_dispatch_body,
        out_type=jax.ShapeDtypeStruct((SLOTS, D_MODEL), jnp.float32),
        mesh=_sc_mesh(),
        scratch_types=_SC_SCRATCH,
    )(x, qf)

    grid_spec = pltpu.PrefetchScalarGridSpec(
        num_scalar_prefetch=1,
        grid=(MAXTILES,),
        in_specs=[
            pl.BlockSpec((BT, D_MODEL), lambda j, te_s: (j, 0)),
            pl.BlockSpec((1, D_FF, D_MODEL), lambda j, te_s: (te_s[0, j], 0, 0)),
            pl.BlockSpec((1, D_FF, D_MODEL), lambda j, te_s: (te_s[0, j], 0, 0)),
            pl.BlockSpec((1, D_MODEL, D_FF), lambda j, te_s: (te_s[0, j], 0, 0)),
        ],
        out_specs=pl.BlockSpec((BT, D_MODEL), lambda j, te_s: (j, 0)),
    )
    out_s = pl.pallas_call(
        _gemm_kernel,
        grid_spec=grid_spec,
        out_shape=jax.ShapeDtypeStruct((SLOTS, D_MODEL), jnp.float32),
    )(te, x_s, w1, v1, w2)

    out = pl.kernel(
        _sc_unpermute_body,
        out_type=jax.ShapeDtypeStruct((T, D_MODEL), jnp.float32),
        mesh=_sc_mesh(),
        scratch_types=_SC_SCRATCH,
    )(out_s, qf)
    return out.reshape(orig_shape)
